# baseline (speedup 1.0000x reference)
import functools

import jax
import jax.numpy as jnp
from jax import lax
from jax.experimental import pallas as pl
from jax.experimental.pallas import tpu as pltpu
from jax.experimental.pallas import tpu_sc as plsc

N = 50000
E = 800000
F_IN = 14
D = 64
M = 64
L = 3
G = 64

N_PAD = 52000
E_PAD = 819200
N_GROUPS = E_PAD // 128
NW = 32
GPW = N_GROUPS // NW
BN = 2000
GRID_N = N_PAD // BN
BE = 8192
GRID_E = E_PAD // BE


def _silu(x):
    return x * jax.nn.sigmoid(x)


def _tck_in_body(x_ref, wp_ref, bp_ref, wa_ref, ba_ref, wb_ref,
                 h_ref, a_ref, b_ref):
    h = jnp.dot(x_ref[...], wp_ref[...],
                preferred_element_type=jnp.float32) + bp_ref[...]
    h_ref[...] = h
    a_ref[...] = jnp.dot(h, wa_ref[...],
                         preferred_element_type=jnp.float32) + ba_ref[...]
    b_ref[...] = jnp.dot(h, wb_ref[...], preferred_element_type=jnp.float32)


def _tck_mid_body(h_ref, mi_ref, wn1a_ref, wn1b_ref, bn1_ref, wn2_ref,
                  bn2_ref, wa_ref, ba_ref, wb_ref, hn_ref, a_ref, b_ref):
    h = h_ref[...]
    t = _silu(jnp.dot(h, wn1a_ref[...], preferred_element_type=jnp.float32)
              + jnp.dot(mi_ref[...], wn1b_ref[...],
                        preferred_element_type=jnp.float32)
              + bn1_ref[...])
    hn = jnp.dot(t, wn2_ref[...],
                 preferred_element_type=jnp.float32) + bn2_ref[...] + h
    hn_ref[...] = hn
    a_ref[...] = jnp.dot(hn, wa_ref[...],
                         preferred_element_type=jnp.float32) + ba_ref[...]
    b_ref[...] = jnp.dot(hn, wb_ref[...], preferred_element_type=jnp.float32)


def _tck_edge_body(p_ref, w_ref, b_ref, m_ref):
    t = _silu(p_ref[...])
    m_ref[...] = _silu(jnp.dot(t, w_ref[...],
                               preferred_element_type=jnp.float32) + b_ref[...])


def _tck_last_body(h_ref, mi_ref, be_ref, bo_ref, wn1a_ref, wn1b_ref,
                   bn1_ref, wn2_ref, bn2_ref, w1_ref, b1_ref, w2_ref, b2_ref,
                   w3_ref, b3_ref, w4_ref, b4_ref,
                   h3_ref, fin_ref, acc_ref):
    i = pl.program_id(0)
    h = h_ref[...]
    t = _silu(jnp.dot(h, wn1a_ref[...], preferred_element_type=jnp.float32)
              + jnp.dot(mi_ref[...], wn1b_ref[...],
                        preferred_element_type=jnp.float32)
              + bn1_ref[...])
    hn = jnp.dot(t, wn2_ref[...],
                 preferred_element_type=jnp.float32) + bn2_ref[...] + h
    h3_ref[...] = hn
    o = _silu(jnp.dot(hn, w1_ref[...],
                      preferred_element_type=jnp.float32) + b1_ref[...])
    o = jnp.dot(o, w2_ref[...], preferred_element_type=jnp.float32) + b2_ref[...]
    gid = lax.broadcasted_iota(jnp.int32, (1, G), 1)
    oh_e = (be_ref[...] == gid).astype(jnp.float32)
    oh_o = (bo_ref[...] == gid).astype(jnp.float32)
    part = (lax.dot_general(oh_e, o[:, :M], (((0,), (0,)), ((), ())),
                            preferred_element_type=jnp.float32)
            + lax.dot_general(oh_o, o[:, M:], (((0,), (0,)), ((), ())),
                              preferred_element_type=jnp.float32))

    @pl.when(i == 0)
    def _():
        acc_ref[...] = jnp.zeros_like(acc_ref)

    acc_ref[...] += part

    @pl.when(i == GRID_N - 1)
    def _():
        og = _silu(jnp.dot(acc_ref[...], w3_ref[...],
                           preferred_element_type=jnp.float32) + b3_ref[...])
        fin_ref[...] = jnp.dot(og, w4_ref[...],
                               preferred_element_type=jnp.float32) + b4_ref[...]


def _row_spec(rows, cols):
    return pl.BlockSpec((rows, cols), lambda i: (i, 0))


def _full_spec(shape):
    nd = len(shape)
    return pl.BlockSpec(shape, lambda i: (0,) * nd)


_N2 = N_PAD // 2
_BN2 = BN // 2
_DP = 2 * M


def _tck_in(x2, wp2, bp2, wa2, ba2, wb2):
    return pl.pallas_call(
        _tck_in_body,
        grid=(GRID_N,),
        in_specs=[_row_spec(_BN2, 2 * F_IN), _full_spec((2 * F_IN, _DP)),
                  _full_spec((1, _DP)), _full_spec((_DP, _DP)),
                  _full_spec((1, _DP)), _full_spec((_DP, _DP))],
        out_specs=[_row_spec(_BN2, _DP), _row_spec(_BN2, _DP),
                   _row_spec(_BN2, _DP)],
        out_shape=[jax.ShapeDtypeStruct((_N2, _DP), jnp.float32),
                   jax.ShapeDtypeStruct((_N2, _DP), jnp.float32),
                   jax.ShapeDtypeStruct((_N2, _DP), jnp.float32)],
    )(x2, wp2, bp2, wa2, ba2, wb2)


def _tck_mid(h2, mi2, wn1a2, wn1b2, bn12, wn22, bn22, wa2, ba2, wb2):
    return pl.pallas_call(
        _tck_mid_body,
        grid=(GRID_N,),
        in_specs=[_row_spec(_BN2, _DP), _row_spec(_BN2, _DP)]
        + [_full_spec((_DP, _DP)), _full_spec((_DP, _DP)),
           _full_spec((1, _DP)), _full_spec((_DP, _DP)),
           _full_spec((1, _DP)), _full_spec((_DP, _DP)),
           _full_spec((1, _DP)), _full_spec((_DP, _DP))],
        out_specs=[_row_spec(_BN2, _DP), _row_spec(_BN2, _DP),
                   _row_spec(_BN2, _DP)],
        out_shape=[jax.ShapeDtypeStruct((_N2, _DP), jnp.float32),
                   jax.ShapeDtypeStruct((_N2, _DP), jnp.float32),
                   jax.ShapeDtypeStruct((_N2, _DP), jnp.float32)],
    )(h2, mi2, wn1a2, wn1b2, bn12, wn22, bn22, wa2, ba2, wb2)


def _tck_edge(p2, w2x2, b2x2):
    return pl.pallas_call(
        _tck_edge_body,
        grid=(GRID_E,),
        in_specs=[_row_spec(BE // 2, 2 * M), _full_spec((2 * M, 2 * M)),
                  _full_spec((1, 2 * M))],
        out_specs=_row_spec(BE // 2, 2 * M),
        out_shape=jax.ShapeDtypeStruct((E_PAD // 2, 2 * M), jnp.float32),
    )(p2, w2x2, b2x2)


def _tck_last(h2, mi2, be, bo, wn1a2, wn1b2, bn12, wn22, bn22,
              w12, b12, w22, b22, w3, b3, w4, b4):
    return pl.pallas_call(
        _tck_last_body,
        grid=(GRID_N,),
        in_specs=[_row_spec(_BN2, _DP), _row_spec(_BN2, _DP),
                  _row_spec(_BN2, 1), _row_spec(_BN2, 1),
                  _full_spec((_DP, _DP)), _full_spec((_DP, _DP)),
                  _full_spec((1, _DP)), _full_spec((_DP, _DP)),
                  _full_spec((1, _DP)), _full_spec((_DP, _DP)),
                  _full_spec((1, _DP)), _full_spec((_DP, _DP)),
                  _full_spec((1, _DP)),
                  _full_spec((D, M)), _full_spec((1, M)),
                  _full_spec((M, 1)), _full_spec((1, 1))],
        out_specs=[_row_spec(_BN2, _DP), _full_spec((G, 1))],
        out_shape=[jax.ShapeDtypeStruct((_N2, _DP), jnp.float32),
                   jax.ShapeDtypeStruct((G, 1), jnp.float32)],
        scratch_shapes=[pltpu.VMEM((G, D), jnp.float32)],
    )(h2, mi2, be, bo, wn1a2, wn1b2, bn12, wn22, bn22,
      w12, b12, w22, b22, w3, b3, w4, b4)


_NGG = 2
_GATHER_ITERS = GPW // (2 * _NGG)
_CG = _NGG * 128

_NGS = 2
_EPS = E_PAD // 16
_GPS = _EPS // 128
_SCATTER_ITERS = _GPS // (2 * _NGS)
_CS = _NGS * 128
_HD = D // 2
_ZR = 416
_NCHUNK = N_PAD // _ZR


_QW = 16
_GQ = 2
_CQ = _GQ * 128
_GPS_G = N_GROUPS // 16
_GATHER_ITERS_Q = _GPS_G // (2 * _GQ)


@functools.lru_cache(maxsize=None)
def _make_sck_gather():
    mesh = plsc.VectorSubcoreMesh(core_axis_name="c", subcore_axis_name="s")
    return functools.partial(
        pl.kernel,
        out_type=jax.ShapeDtypeStruct((E_PAD, M), jnp.float32),
        mesh=mesh,
        scratch_types=[
            pltpu.VMEM((2 * _GQ, 128), jnp.int32),
            pltpu.VMEM((2 * _GQ, 128), jnp.int32),
            pltpu.VMEM((2 * _CQ, _QW), jnp.float32),
            pltpu.VMEM((2 * _CQ, _QW), jnp.float32),
            pltpu.VMEM_SHARED((N_PAD, _QW), jnp.float32),
            pltpu.VMEM_SHARED((N_PAD, _QW), jnp.float32),
            pltpu.SemaphoreType.DMA,
            pltpu.SemaphoreType.DMA,
            pltpu.SemaphoreType.DMA,
            pltpu.SemaphoreType.DMA,
            pltpu.SemaphoreType.DMA,
            pltpu.SemaphoreType.DMA,
        ],
        compiler_params=pltpu.CompilerParams(use_tc_tiling_on_sc=False),
    )(_sck_gather_body)


def _sck_gather(A, B, dst_p, src_p):
    return _make_sck_gather()(A, B, dst_p, src_p)


def _sck_gather_body(a_hbm, b_hbm, dst_hbm, src_hbm, p_hbm,
                     idxd_v, idxs_v, bufa_v, bufb_v, a_sp, b_sp,
                     is_e, is_o, g_e, g_o, w_e, w_o):
    c = lax.axis_index("c")
    s = lax.axis_index("s")
    gbase = s * _GPS_G

    def fire_idx(pair):
        goff_e = pl.multiple_of(gbase + (2 * pair) * _GQ, _GQ)
        goff_o = pl.multiple_of(gbase + (2 * pair + 1) * _GQ, _GQ)
        pltpu.async_copy(dst_hbm.at[pl.ds(goff_e, _GQ)],
                         idxd_v.at[pl.ds(0, _GQ)], is_e)
        pltpu.async_copy(src_hbm.at[pl.ds(goff_e, _GQ)],
                         idxs_v.at[pl.ds(0, _GQ)], is_e)
        pltpu.async_copy(dst_hbm.at[pl.ds(goff_o, _GQ)],
                         idxd_v.at[pl.ds(_GQ, _GQ)], is_o)
        pltpu.async_copy(src_hbm.at[pl.ds(goff_o, _GQ)],
                         idxs_v.at[pl.ds(_GQ, _GQ)], is_o)

    def wait_idx():
        pltpu.make_async_copy(dst_hbm.at[pl.ds(0, _GQ)],
                              idxd_v.at[pl.ds(0, _GQ)], is_e).wait()
        pltpu.make_async_copy(src_hbm.at[pl.ds(0, _GQ)],
                              idxs_v.at[pl.ds(0, _GQ)], is_e).wait()
        pltpu.make_async_copy(dst_hbm.at[pl.ds(0, _GQ)],
                              idxd_v.at[pl.ds(_GQ, _GQ)], is_o).wait()
        pltpu.make_async_copy(src_hbm.at[pl.ds(0, _GQ)],
                              idxs_v.at[pl.ds(_GQ, _GQ)], is_o).wait()

    def wait_writes(col0):
        pltpu.make_async_copy(
            bufa_v.at[pl.ds(0, _CQ)],
            p_hbm.at[pl.ds(0, _CQ), pl.ds(col0, _QW)], w_e).wait()
        pltpu.make_async_copy(
            bufa_v.at[pl.ds(_CQ, _CQ)],
            p_hbm.at[pl.ds(_CQ, _CQ), pl.ds(col0, _QW)], w_o).wait()

    for q in range(2):
        col0 = c * (D // 2) + q * _QW
        for k in range(8):
            cid = s + 16 * k

            @pl.when(cid < _NCHUNK)
            def _():
                pltpu.sync_copy(
                    a_hbm.at[pl.ds(cid * _ZR, _ZR), pl.ds(col0, _QW)],
                    a_sp.at[pl.ds(cid * _ZR, _ZR)])
                pltpu.sync_copy(
                    b_hbm.at[pl.ds(cid * _ZR, _ZR), pl.ds(col0, _QW)],
                    b_sp.at[pl.ds(cid * _ZR, _ZR)])

        plsc.subcore_barrier()
        fire_idx(0)

        def iteration(k, carry):
            goff_e = pl.multiple_of(gbase + (2 * k) * _GQ, _GQ)
            goff_o = pl.multiple_of(gbase + (2 * k + 1) * _GQ, _GQ)
            eoff_e = pl.multiple_of(goff_e * 128, _CQ)
            eoff_o = pl.multiple_of(goff_o * 128, _CQ)

            @pl.when(k > 0)
            def _():
                wait_writes(col0)

            wait_idx()
            dg_e = []
            for j in range(_GQ):
                dg_e.append(pltpu.async_copy(
                    a_sp.at[idxd_v.at[j]],
                    bufa_v.at[pl.ds(j * 128, 128)], g_e))
                dg_e.append(pltpu.async_copy(
                    b_sp.at[idxs_v.at[j]],
                    bufb_v.at[pl.ds(j * 128, 128)], g_e))
            dg_o = []
            for j in range(_GQ, 2 * _GQ):
                dg_o.append(pltpu.async_copy(
                    a_sp.at[idxd_v.at[j]],
                    bufa_v.at[pl.ds(j * 128, 128)], g_o))
                dg_o.append(pltpu.async_copy(
                    b_sp.at[idxs_v.at[j]],
                    bufb_v.at[pl.ds(j * 128, 128)], g_o))

            def add_rows(base):
                def add_row(r, cc):
                    bufa_v[base + r, pl.ds(0, 16)] += \
                        bufb_v[base + r, pl.ds(0, 16)]
                    return cc
                lax.fori_loop(0, _CQ, add_row, 0, unroll=8)

            for d in dg_e:
                d.wait()
            add_rows(0)
            pltpu.async_copy(
                bufa_v.at[pl.ds(0, _CQ)],
                p_hbm.at[pl.ds(eoff_e, _CQ), pl.ds(col0, _QW)], w_e)
            for d in dg_o:
                d.wait()

            @pl.when(k + 1 < _GATHER_ITERS_Q)
            def _():
                fire_idx(k + 1)

            add_rows(_CQ)
            pltpu.async_copy(
                bufa_v.at[pl.ds(_CQ, _CQ)],
                p_hbm.at[pl.ds(eoff_o, _CQ), pl.ds(col0, _QW)], w_o)
            return carry

        lax.fori_loop(0, _GATHER_ITERS_Q, iteration, 0)
        wait_writes(col0)
        plsc.subcore_barrier()


@functools.lru_cache(maxsize=None)
def _make_sck_scatter():
    mesh = plsc.VectorSubcoreMesh(core_axis_name="c", subcore_axis_name="s")
    return functools.partial(
        pl.kernel,
        out_type=jax.ShapeDtypeStruct((N_PAD, D), jnp.float32),
        mesh=mesh,
        scratch_types=[
            pltpu.VMEM((2 * _NGS, 128), jnp.int32),
            pltpu.VMEM((2 * _CS, _HD), jnp.float32),
            pltpu.VMEM_SHARED((N_PAD, _HD), jnp.float32),
            pltpu.SemaphoreType.DMA,
            pltpu.SemaphoreType.DMA,
            pltpu.SemaphoreType.DMA,
            pltpu.SemaphoreType.DMA,
        ],
        compiler_params=pltpu.CompilerParams(use_tc_tiling_on_sc=False),
    )(_sck_scatter_body)


def _sck_scatter(m, dst_p):
    return _make_sck_scatter()(m, dst_p)


def _sck_scatter_body(m_hbm, dst_hbm, mi_hbm, idx_v, rows_v, acc_sh,
                      is_e, is_o, m_e, m_o):
    c = lax.axis_index("c")
    s = lax.axis_index("s")
    col0 = c * _HD

    def zrow(r, carry):
        for k in range(_HD // 16):
            rows_v[r, pl.ds(k * 16, 16)] = jnp.zeros((16,), jnp.float32)
        return carry

    lax.fori_loop(0, _ZR, zrow, 0)
    for k in range(8):
        cid = s + 16 * k

        @pl.when(cid < _NCHUNK)
        def _():
            pltpu.sync_copy(rows_v.at[pl.ds(0, _ZR)],
                            acc_sh.at[pl.ds(cid * _ZR, _ZR)])

    plsc.subcore_barrier()

    gbase = s * _GPS

    def iteration(k, carry):
        goff_e = pl.multiple_of(gbase + (2 * k) * _NGS, _NGS)
        goff_o = pl.multiple_of(gbase + (2 * k + 1) * _NGS, _NGS)
        eoff_e = pl.multiple_of(goff_e * 128, _CS)
        eoff_o = pl.multiple_of(goff_o * 128, _CS)
        di_e = pltpu.async_copy(dst_hbm.at[pl.ds(goff_e, _NGS)],
                                idx_v.at[pl.ds(0, _NGS)], is_e)
        dm_e = pltpu.async_copy(
            m_hbm.at[pl.ds(eoff_e, _CS), pl.ds(col0, _HD)],
            rows_v.at[pl.ds(0, _CS)], m_e)
        di_o = pltpu.async_copy(dst_hbm.at[pl.ds(goff_o, _NGS)],
                                idx_v.at[pl.ds(_NGS, _NGS)], is_o)
        dm_o = pltpu.async_copy(
            m_hbm.at[pl.ds(eoff_o, _CS), pl.ds(col0, _HD)],
            rows_v.at[pl.ds(_CS, _CS)], m_o)
        di_e.wait()
        dm_e.wait()
        for j in range(_NGS):
            pltpu.sync_copy(rows_v.at[pl.ds(j * 128, 128)],
                            acc_sh.at[idx_v.at[j]], add=True)
        di_o.wait()
        dm_o.wait()
        for j in range(_NGS, 2 * _NGS):
            pltpu.sync_copy(rows_v.at[pl.ds(j * 128, 128)],
                            acc_sh.at[idx_v.at[j]], add=True)
        return carry

    lax.fori_loop(0, _SCATTER_ITERS, iteration, 0)
    plsc.subcore_barrier()

    for k in range(8):
        cid = s + 16 * k

        @pl.when(cid < _NCHUNK)
        def _():
            pltpu.sync_copy(acc_sh.at[pl.ds(cid * _ZR, _ZR)],
                            mi_hbm.at[pl.ds(cid * _ZR, _ZR),
                                      pl.ds(col0, _HD)])


def kernel(x, pos, edge_index, batch, Wp, bp, We1, be1, We2, be2,
           Wn1, bn1, Wn2, bn2, W1, b1, W2, b2, W3, b3, W4, b4):
    f32 = jnp.float32

    def bd(w):
        z = jnp.zeros(w.shape, f32)
        return jnp.concatenate([
            jnp.concatenate([w, z], axis=-1),
            jnp.concatenate([z, w], axis=-1)], axis=-2)

    def dup(b):
        return jnp.concatenate([b, b], axis=-1)[..., None, :]

    x_p = jnp.pad(x, ((0, N_PAD - N), (0, 0)))
    x2 = x_p.reshape(_N2, 2 * F_IN)
    batch_p = jnp.pad(batch, (0, N_PAD - N), constant_values=G)
    be_ = batch_p[0::2].reshape(_N2, 1)
    bo_ = batch_p[1::2].reshape(_N2, 1)
    npad = E_PAD - E
    pad_rows = N + (jnp.arange(npad, dtype=jnp.int32) % 32)
    src_p = jnp.concatenate([edge_index[0], pad_rows]).reshape(N_GROUPS, 128)
    dst_p = jnp.concatenate([edge_index[1], pad_rows]).reshape(N_GROUPS, 128)

    Wa2 = bd(We1[:, :D, :])
    Wb2 = bd(We1[:, D:, :])
    Wn1a2 = bd(Wn1[:, :D, :])
    Wn1b2 = bd(Wn1[:, D:, :])
    Wn22 = bd(Wn2)
    w2x2 = bd(We2)
    Wp2 = bd(Wp)
    W12 = bd(W1)
    W22 = bd(W2)
    ba2 = dup(be1)
    be2_2x = dup(be2)
    bn12 = dup(bn1)
    bn22 = dup(bn2)
    bp2 = dup(bp)
    b12 = dup(b1)
    b22 = dup(b2)
    b3_2 = b3.reshape(1, M)
    b4_2 = b4.reshape(1, 1)

    h2, A2, B2 = _tck_in(x2, Wp2, bp2, Wa2[0], ba2[0], Wb2[0])
    for l in range(L):
        P = _sck_gather(A2.reshape(N_PAD, M), B2.reshape(N_PAD, M),
                        dst_p, src_p)
        m2 = _tck_edge(P.reshape(E_PAD // 2, 2 * M), w2x2[l], be2_2x[l])
        mi = _sck_scatter(m2.reshape(E_PAD, M), dst_p)
        mi2 = mi.reshape(_N2, _DP)
        if l + 1 < L:
            h2, A2, B2 = _tck_mid(h2, mi2, Wn1a2[l], Wn1b2[l], bn12[l],
                                  Wn22[l], bn22[l], Wa2[l + 1], ba2[l + 1],
                                  Wb2[l + 1])
        else:
            h3_2, fin = _tck_last(h2, mi2, be_, bo_, Wn1a2[l], Wn1b2[l],
                                  bn12[l], Wn22[l], bn22[l], W12, b12,
                                  W22, b22, W3, b3_2, W4, b4_2)
    h3 = h3_2.reshape(N_PAD, D)[:N]
    return (fin.reshape(-1), h3.astype(f32))


# speedup vs baseline: 1.2556x; 1.2556x over previous
import functools

import jax
import jax.numpy as jnp
from jax import lax
from jax.experimental import pallas as pl
from jax.experimental.pallas import tpu as pltpu
from jax.experimental.pallas import tpu_sc as plsc

N = 50000
E = 800000
F_IN = 14
D = 64
M = 64
L = 3
G = 64

N_PAD = 52000
E_PAD = 819200
N_GROUPS = E_PAD // 128
NW = 32
GPW = N_GROUPS // NW
BN = 2000
GRID_N = N_PAD // BN
BE = 8192
GRID_E = E_PAD // BE


def _silu(x):
    return x * jax.nn.sigmoid(x)


def _tck_in_body(x_ref, wp_ref, bp_ref, wa_ref, ba_ref, wb_ref,
                 h_ref, a_ref, b_ref):
    h = jnp.dot(x_ref[...], wp_ref[...],
                preferred_element_type=jnp.float32) + bp_ref[...]
    h_ref[...] = h
    a_ref[...] = jnp.dot(h, wa_ref[...],
                         preferred_element_type=jnp.float32) + ba_ref[...]
    b_ref[...] = jnp.dot(h, wb_ref[...], preferred_element_type=jnp.float32)


def _tck_mid_body(h_ref, mi_ref, wn1a_ref, wn1b_ref, bn1_ref, wn2_ref,
                  bn2_ref, wa_ref, ba_ref, wb_ref, hn_ref, a_ref, b_ref):
    h = h_ref[...]
    t = _silu(jnp.dot(h, wn1a_ref[...], preferred_element_type=jnp.float32)
              + jnp.dot(mi_ref[...], wn1b_ref[...],
                        preferred_element_type=jnp.float32)
              + bn1_ref[...])
    hn = jnp.dot(t, wn2_ref[...],
                 preferred_element_type=jnp.float32) + bn2_ref[...] + h
    hn_ref[...] = hn
    a_ref[...] = jnp.dot(hn, wa_ref[...],
                         preferred_element_type=jnp.float32) + ba_ref[...]
    b_ref[...] = jnp.dot(hn, wb_ref[...], preferred_element_type=jnp.float32)


def _tck_edge_body(p_ref, w_ref, b_ref, m_ref):
    t = _silu(p_ref[...])
    m_ref[...] = _silu(jnp.dot(t, w_ref[...],
                               preferred_element_type=jnp.float32) + b_ref[...])


def _tck_last_body(h_ref, mi_ref, be_ref, bo_ref, wn1a_ref, wn1b_ref,
                   bn1_ref, wn2_ref, bn2_ref, w1_ref, b1_ref, w2_ref, b2_ref,
                   w3_ref, b3_ref, w4_ref, b4_ref,
                   h3_ref, fin_ref, acc_ref):
    i = pl.program_id(0)
    h = h_ref[...]
    t = _silu(jnp.dot(h, wn1a_ref[...], preferred_element_type=jnp.float32)
              + jnp.dot(mi_ref[...], wn1b_ref[...],
                        preferred_element_type=jnp.float32)
              + bn1_ref[...])
    hn = jnp.dot(t, wn2_ref[...],
                 preferred_element_type=jnp.float32) + bn2_ref[...] + h
    h3_ref[...] = hn
    o = _silu(jnp.dot(hn, w1_ref[...],
                      preferred_element_type=jnp.float32) + b1_ref[...])
    o = jnp.dot(o, w2_ref[...], preferred_element_type=jnp.float32) + b2_ref[...]
    gid = lax.broadcasted_iota(jnp.int32, (1, G), 1)
    oh_e = (be_ref[...] == gid).astype(jnp.float32)
    oh_o = (bo_ref[...] == gid).astype(jnp.float32)
    part = (lax.dot_general(oh_e, o[:, :M], (((0,), (0,)), ((), ())),
                            preferred_element_type=jnp.float32)
            + lax.dot_general(oh_o, o[:, M:], (((0,), (0,)), ((), ())),
                              preferred_element_type=jnp.float32))

    @pl.when(i == 0)
    def _():
        acc_ref[...] = jnp.zeros_like(acc_ref)

    acc_ref[...] += part

    @pl.when(i == GRID_N - 1)
    def _():
        og = _silu(jnp.dot(acc_ref[...], w3_ref[...],
                           preferred_element_type=jnp.float32) + b3_ref[...])
        fin_ref[...] = jnp.dot(og, w4_ref[...],
                               preferred_element_type=jnp.float32) + b4_ref[...]


def _row_spec(rows, cols):
    return pl.BlockSpec((rows, cols), lambda i: (i, 0))


def _full_spec(shape):
    nd = len(shape)
    return pl.BlockSpec(shape, lambda i: (0,) * nd)


_N2 = N_PAD // 2
_BN2 = BN // 2
_DP = 2 * M


def _tck_in(x2, wp2, bp2, wa2, ba2, wb2):
    return pl.pallas_call(
        _tck_in_body,
        grid=(GRID_N,),
        in_specs=[_row_spec(_BN2, 2 * F_IN), _full_spec((2 * F_IN, _DP)),
                  _full_spec((1, _DP)), _full_spec((_DP, _DP)),
                  _full_spec((1, _DP)), _full_spec((_DP, _DP))],
        out_specs=[_row_spec(_BN2, _DP), _row_spec(_BN2, _DP),
                   _row_spec(_BN2, _DP)],
        out_shape=[jax.ShapeDtypeStruct((_N2, _DP), jnp.float32),
                   jax.ShapeDtypeStruct((_N2, _DP), jnp.float32),
                   jax.ShapeDtypeStruct((_N2, _DP), jnp.float32)],
    )(x2, wp2, bp2, wa2, ba2, wb2)


def _tck_mid(h2, mi2, wn1a2, wn1b2, bn12, wn22, bn22, wa2, ba2, wb2):
    return pl.pallas_call(
        _tck_mid_body,
        grid=(GRID_N,),
        in_specs=[_row_spec(_BN2, _DP), _row_spec(_BN2, _DP)]
        + [_full_spec((_DP, _DP)), _full_spec((_DP, _DP)),
           _full_spec((1, _DP)), _full_spec((_DP, _DP)),
           _full_spec((1, _DP)), _full_spec((_DP, _DP)),
           _full_spec((1, _DP)), _full_spec((_DP, _DP))],
        out_specs=[_row_spec(_BN2, _DP), _row_spec(_BN2, _DP),
                   _row_spec(_BN2, _DP)],
        out_shape=[jax.ShapeDtypeStruct((_N2, _DP), jnp.float32),
                   jax.ShapeDtypeStruct((_N2, _DP), jnp.float32),
                   jax.ShapeDtypeStruct((_N2, _DP), jnp.float32)],
    )(h2, mi2, wn1a2, wn1b2, bn12, wn22, bn22, wa2, ba2, wb2)


def _tck_edge(p2, w2x2, b2x2):
    return pl.pallas_call(
        _tck_edge_body,
        grid=(GRID_E,),
        in_specs=[_row_spec(BE // 2, 2 * M), _full_spec((2 * M, 2 * M)),
                  _full_spec((1, 2 * M))],
        out_specs=_row_spec(BE // 2, 2 * M),
        out_shape=jax.ShapeDtypeStruct((E_PAD // 2, 2 * M), jnp.float32),
    )(p2, w2x2, b2x2)


def _tck_last(h2, mi2, be, bo, wn1a2, wn1b2, bn12, wn22, bn22,
              w12, b12, w22, b22, w3, b3, w4, b4):
    return pl.pallas_call(
        _tck_last_body,
        grid=(GRID_N,),
        in_specs=[_row_spec(_BN2, _DP), _row_spec(_BN2, _DP),
                  _row_spec(_BN2, 1), _row_spec(_BN2, 1),
                  _full_spec((_DP, _DP)), _full_spec((_DP, _DP)),
                  _full_spec((1, _DP)), _full_spec((_DP, _DP)),
                  _full_spec((1, _DP)), _full_spec((_DP, _DP)),
                  _full_spec((1, _DP)), _full_spec((_DP, _DP)),
                  _full_spec((1, _DP)),
                  _full_spec((D, M)), _full_spec((1, M)),
                  _full_spec((M, 1)), _full_spec((1, 1))],
        out_specs=[_row_spec(_BN2, _DP), _full_spec((G, 1))],
        out_shape=[jax.ShapeDtypeStruct((_N2, _DP), jnp.float32),
                   jax.ShapeDtypeStruct((G, 1), jnp.float32)],
        scratch_shapes=[pltpu.VMEM((G, D), jnp.float32)],
    )(h2, mi2, be, bo, wn1a2, wn1b2, bn12, wn22, bn22,
      w12, b12, w22, b22, w3, b3, w4, b4)


_NGG = 2
_GATHER_ITERS = GPW // (2 * _NGG)
_CG = _NGG * 128

_NGS = 2
_EPS = E_PAD // 16
_GPS = _EPS // 128
_SCATTER_ITERS = _GPS // (2 * _NGS)
_CS = _NGS * 128
_HD = D // 2
_ZR = 416
_NCHUNK = N_PAD // _ZR


@functools.lru_cache(maxsize=None)
def _make_sck_gather():
    mesh = plsc.VectorSubcoreMesh(core_axis_name="c", subcore_axis_name="s")
    return functools.partial(
        pl.kernel,
        out_type=jax.ShapeDtypeStruct((E_PAD, M), jnp.float32),
        mesh=mesh,
        scratch_types=[
            pltpu.VMEM((2 * _NGG, 128), jnp.int32),
            pltpu.VMEM((2 * _NGG, 128), jnp.int32),
            pltpu.VMEM((2 * _CG, M), jnp.float32),
            pltpu.VMEM((2 * _CG, M), jnp.float32),
            pltpu.SemaphoreType.DMA,
            pltpu.SemaphoreType.DMA,
            pltpu.SemaphoreType.DMA,
            pltpu.SemaphoreType.DMA,
            pltpu.SemaphoreType.DMA,
            pltpu.SemaphoreType.DMA,
        ],
        compiler_params=pltpu.CompilerParams(use_tc_tiling_on_sc=False),
    )(_sck_gather_body)


def _sck_gather(A, B, dst_p, src_p):
    return _make_sck_gather()(A, B, dst_p, src_p)


def _sck_gather_body(a_hbm, b_hbm, dst_hbm, src_hbm, p_hbm,
                     idxd_v, idxs_v, bufa_v, bufb_v,
                     is_e, is_o, g_e, g_o, w_e, w_o):
    wid = lax.axis_index("s") * 2 + lax.axis_index("c")
    gbase = wid * GPW

    def fire_idx(pair):
        goff_e = pl.multiple_of(gbase + (2 * pair) * _NGG, _NGG)
        goff_o = pl.multiple_of(gbase + (2 * pair + 1) * _NGG, _NGG)
        pltpu.async_copy(dst_hbm.at[pl.ds(goff_e, _NGG)],
                         idxd_v.at[pl.ds(0, _NGG)], is_e)
        pltpu.async_copy(src_hbm.at[pl.ds(goff_e, _NGG)],
                         idxs_v.at[pl.ds(0, _NGG)], is_e)
        pltpu.async_copy(dst_hbm.at[pl.ds(goff_o, _NGG)],
                         idxd_v.at[pl.ds(_NGG, _NGG)], is_o)
        pltpu.async_copy(src_hbm.at[pl.ds(goff_o, _NGG)],
                         idxs_v.at[pl.ds(_NGG, _NGG)], is_o)

    def wait_idx():
        pltpu.make_async_copy(dst_hbm.at[pl.ds(0, _NGG)],
                              idxd_v.at[pl.ds(0, _NGG)], is_e).wait()
        pltpu.make_async_copy(src_hbm.at[pl.ds(0, _NGG)],
                              idxs_v.at[pl.ds(0, _NGG)], is_e).wait()
        pltpu.make_async_copy(dst_hbm.at[pl.ds(0, _NGG)],
                              idxd_v.at[pl.ds(_NGG, _NGG)], is_o).wait()
        pltpu.make_async_copy(src_hbm.at[pl.ds(0, _NGG)],
                              idxs_v.at[pl.ds(_NGG, _NGG)], is_o).wait()

    def wait_writes():
        pltpu.make_async_copy(bufa_v.at[pl.ds(0, _CG)],
                              p_hbm.at[pl.ds(0, _CG)], w_e).wait()
        pltpu.make_async_copy(bufa_v.at[pl.ds(_CG, _CG)],
                              p_hbm.at[pl.ds(_CG, _CG)], w_o).wait()

    fire_idx(0)

    def iteration(k, carry):
        goff_e = pl.multiple_of(gbase + (2 * k) * _NGG, _NGG)
        goff_o = pl.multiple_of(gbase + (2 * k + 1) * _NGG, _NGG)
        eoff_e = pl.multiple_of(goff_e * 128, _CG)
        eoff_o = pl.multiple_of(goff_o * 128, _CG)

        @pl.when(k > 0)
        def _():
            wait_writes()

        wait_idx()
        dg_e = []
        for j in range(_NGG):
            dg_e.append(pltpu.async_copy(
                a_hbm.at[idxd_v.at[j]],
                bufa_v.at[pl.ds(j * 128, 128)], g_e))
            dg_e.append(pltpu.async_copy(
                b_hbm.at[idxs_v.at[j]],
                bufb_v.at[pl.ds(j * 128, 128)], g_e))
        dg_o = []
        for j in range(_NGG, 2 * _NGG):
            dg_o.append(pltpu.async_copy(
                a_hbm.at[idxd_v.at[j]],
                bufa_v.at[pl.ds(j * 128, 128)], g_o))
            dg_o.append(pltpu.async_copy(
                b_hbm.at[idxs_v.at[j]],
                bufb_v.at[pl.ds(j * 128, 128)], g_o))

        def add_rows(base):
            def add_row(r, c):
                for kk in range(M // 16):
                    sl = pl.ds(kk * 16, 16)
                    bufa_v[base + r, sl] += bufb_v[base + r, sl]
                return c
            lax.fori_loop(0, _CG, add_row, 0, unroll=4)

        for d in dg_e:
            d.wait()
        add_rows(0)
        pltpu.async_copy(bufa_v.at[pl.ds(0, _CG)],
                         p_hbm.at[pl.ds(eoff_e, _CG)], w_e)
        for d in dg_o:
            d.wait()

        @pl.when(k + 1 < _GATHER_ITERS)
        def _():
            fire_idx(k + 1)

        add_rows(_CG)
        pltpu.async_copy(bufa_v.at[pl.ds(_CG, _CG)],
                         p_hbm.at[pl.ds(eoff_o, _CG)], w_o)
        return carry

    lax.fori_loop(0, _GATHER_ITERS, iteration, 0)
    wait_writes()


@functools.lru_cache(maxsize=None)
def _make_sck_scatter():
    mesh = plsc.VectorSubcoreMesh(core_axis_name="c", subcore_axis_name="s")
    return functools.partial(
        pl.kernel,
        out_type=jax.ShapeDtypeStruct((N_PAD, D), jnp.float32),
        mesh=mesh,
        scratch_types=[
            pltpu.VMEM((2 * _NGS, 128), jnp.int32),
            pltpu.VMEM((2 * _CS, _HD), jnp.float32),
            pltpu.VMEM_SHARED((N_PAD, _HD), jnp.float32),
            pltpu.SemaphoreType.DMA,
            pltpu.SemaphoreType.DMA,
            pltpu.SemaphoreType.DMA,
            pltpu.SemaphoreType.DMA,
        ],
        compiler_params=pltpu.CompilerParams(use_tc_tiling_on_sc=False),
    )(_sck_scatter_body)


def _sck_scatter(m, dst_p):
    return _make_sck_scatter()(m, dst_p)


def _sck_scatter_body(m_hbm, dst_hbm, mi_hbm, idx_v, rows_v, acc_sh,
                      is_e, is_o, m_e, m_o):
    c = lax.axis_index("c")
    s = lax.axis_index("s")
    col0 = c * _HD

    def zrow(r, carry):
        for k in range(_HD // 16):
            rows_v[r, pl.ds(k * 16, 16)] = jnp.zeros((16,), jnp.float32)
        return carry

    lax.fori_loop(0, _ZR, zrow, 0)
    for k in range(8):
        cid = s + 16 * k

        @pl.when(cid < _NCHUNK)
        def _():
            pltpu.sync_copy(rows_v.at[pl.ds(0, _ZR)],
                            acc_sh.at[pl.ds(cid * _ZR, _ZR)])

    plsc.subcore_barrier()

    gbase = s * _GPS

    def iteration(k, carry):
        goff_e = pl.multiple_of(gbase + (2 * k) * _NGS, _NGS)
        goff_o = pl.multiple_of(gbase + (2 * k + 1) * _NGS, _NGS)
        eoff_e = pl.multiple_of(goff_e * 128, _CS)
        eoff_o = pl.multiple_of(goff_o * 128, _CS)
        di_e = pltpu.async_copy(dst_hbm.at[pl.ds(goff_e, _NGS)],
                                idx_v.at[pl.ds(0, _NGS)], is_e)
        dm_e = pltpu.async_copy(
            m_hbm.at[pl.ds(eoff_e, _CS), pl.ds(col0, _HD)],
            rows_v.at[pl.ds(0, _CS)], m_e)
        di_o = pltpu.async_copy(dst_hbm.at[pl.ds(goff_o, _NGS)],
                                idx_v.at[pl.ds(_NGS, _NGS)], is_o)
        dm_o = pltpu.async_copy(
            m_hbm.at[pl.ds(eoff_o, _CS), pl.ds(col0, _HD)],
            rows_v.at[pl.ds(_CS, _CS)], m_o)
        di_e.wait()
        dm_e.wait()
        for j in range(_NGS):
            pltpu.sync_copy(rows_v.at[pl.ds(j * 128, 128)],
                            acc_sh.at[idx_v.at[j]], add=True)
        di_o.wait()
        dm_o.wait()
        for j in range(_NGS, 2 * _NGS):
            pltpu.sync_copy(rows_v.at[pl.ds(j * 128, 128)],
                            acc_sh.at[idx_v.at[j]], add=True)
        return carry

    lax.fori_loop(0, _SCATTER_ITERS, iteration, 0)
    plsc.subcore_barrier()

    for k in range(8):
        cid = s + 16 * k

        @pl.when(cid < _NCHUNK)
        def _():
            pltpu.sync_copy(acc_sh.at[pl.ds(cid * _ZR, _ZR)],
                            mi_hbm.at[pl.ds(cid * _ZR, _ZR),
                                      pl.ds(col0, _HD)])


def kernel(x, pos, edge_index, batch, Wp, bp, We1, be1, We2, be2,
           Wn1, bn1, Wn2, bn2, W1, b1, W2, b2, W3, b3, W4, b4):
    f32 = jnp.float32

    def bd(w):
        z = jnp.zeros(w.shape, f32)
        return jnp.concatenate([
            jnp.concatenate([w, z], axis=-1),
            jnp.concatenate([z, w], axis=-1)], axis=-2)

    def dup(b):
        return jnp.concatenate([b, b], axis=-1)[..., None, :]

    x_p = jnp.pad(x, ((0, N_PAD - N), (0, 0)))
    x2 = x_p.reshape(_N2, 2 * F_IN)
    batch_p = jnp.pad(batch, (0, N_PAD - N), constant_values=G)
    be_ = batch_p[0::2].reshape(_N2, 1)
    bo_ = batch_p[1::2].reshape(_N2, 1)
    npad = E_PAD - E
    pad_rows = N + (jnp.arange(npad, dtype=jnp.int32) % 32)
    src_p = jnp.concatenate([edge_index[0], pad_rows]).reshape(N_GROUPS, 128)
    dst_p = jnp.concatenate([edge_index[1], pad_rows]).reshape(N_GROUPS, 128)

    Wa2 = bd(We1[:, :D, :])
    Wb2 = bd(We1[:, D:, :])
    Wn1a2 = bd(Wn1[:, :D, :])
    Wn1b2 = bd(Wn1[:, D:, :])
    Wn22 = bd(Wn2)
    w2x2 = bd(We2)
    Wp2 = bd(Wp)
    W12 = bd(W1)
    W22 = bd(W2)
    ba2 = dup(be1)
    be2_2x = dup(be2)
    bn12 = dup(bn1)
    bn22 = dup(bn2)
    bp2 = dup(bp)
    b12 = dup(b1)
    b22 = dup(b2)
    b3_2 = b3.reshape(1, M)
    b4_2 = b4.reshape(1, 1)

    h2, A2, B2 = _tck_in(x2, Wp2, bp2, Wa2[0], ba2[0], Wb2[0])
    for l in range(L):
        P = _sck_gather(A2.reshape(N_PAD, M), B2.reshape(N_PAD, M),
                        dst_p, src_p)
        m2 = _tck_edge(P.reshape(E_PAD // 2, 2 * M), w2x2[l], be2_2x[l])
        mi = _sck_scatter(m2.reshape(E_PAD, M), dst_p)
        mi2 = mi.reshape(_N2, _DP)
        if l + 1 < L:
            h2, A2, B2 = _tck_mid(h2, mi2, Wn1a2[l], Wn1b2[l], bn12[l],
                                  Wn22[l], bn22[l], Wa2[l + 1], ba2[l + 1],
                                  Wb2[l + 1])
        else:
            h3_2, fin = _tck_last(h2, mi2, be_, bo_, Wn1a2[l], Wn1b2[l],
                                  bn12[l], Wn22[l], bn22[l], W12, b12,
                                  W22, b22, W3, b3_2, W4, b4_2)
    h3 = h3_2.reshape(N_PAD, D)[:N]
    return (fin.reshape(-1), h3.astype(f32))


# speedup vs baseline: 1.2789x; 1.0186x over previous
import functools

import jax
import jax.numpy as jnp
from jax import lax
from jax.experimental import pallas as pl
from jax.experimental.pallas import tpu as pltpu
from jax.experimental.pallas import tpu_sc as plsc

N = 50000
E = 800000
F_IN = 14
D = 64
M = 64
L = 3
G = 64

N_PAD = 52000
E_PAD = 819200
N_GROUPS = E_PAD // 128
NW = 32
GPW = N_GROUPS // NW
BN = 2000
GRID_N = N_PAD // BN
BE = 8192
GRID_E = E_PAD // BE


def _silu(x):
    return x * jax.nn.sigmoid(x)


def _tck_in_body(x_ref, wp_ref, bp_ref, wa_ref, ba_ref, wb_ref,
                 h_ref, a_ref, b_ref):
    h = jnp.dot(x_ref[...], wp_ref[...],
                preferred_element_type=jnp.float32) + bp_ref[...]
    h_ref[...] = h
    a_ref[...] = jnp.dot(h, wa_ref[...],
                         preferred_element_type=jnp.float32) + ba_ref[...]
    b_ref[...] = jnp.dot(h, wb_ref[...], preferred_element_type=jnp.float32)


def _tck_mid_body(h_ref, mi_ref, wn1a_ref, wn1b_ref, bn1_ref, wn2_ref,
                  bn2_ref, wa_ref, ba_ref, wb_ref, hn_ref, a_ref, b_ref):
    h = h_ref[...]
    t = _silu(jnp.dot(h, wn1a_ref[...], preferred_element_type=jnp.float32)
              + jnp.dot(mi_ref[...], wn1b_ref[...],
                        preferred_element_type=jnp.float32)
              + bn1_ref[...])
    hn = jnp.dot(t, wn2_ref[...],
                 preferred_element_type=jnp.float32) + bn2_ref[...] + h
    hn_ref[...] = hn
    a_ref[...] = jnp.dot(hn, wa_ref[...],
                         preferred_element_type=jnp.float32) + ba_ref[...]
    b_ref[...] = jnp.dot(hn, wb_ref[...], preferred_element_type=jnp.float32)


def _tck_edge_body(p_ref, w_ref, b_ref, m_ref):
    t = _silu(p_ref[...])
    m_ref[...] = _silu(jnp.dot(t, w_ref[...],
                               preferred_element_type=jnp.float32) + b_ref[...])


def _tck_last_body(h_ref, mi_ref, be_ref, bo_ref, wn1a_ref, wn1b_ref,
                   bn1_ref, wn2_ref, bn2_ref, w1_ref, b1_ref, w2_ref, b2_ref,
                   w3_ref, b3_ref, w4_ref, b4_ref,
                   h3_ref, fin_ref, acc_ref):
    i = pl.program_id(0)
    h = h_ref[...]
    t = _silu(jnp.dot(h, wn1a_ref[...], preferred_element_type=jnp.float32)
              + jnp.dot(mi_ref[...], wn1b_ref[...],
                        preferred_element_type=jnp.float32)
              + bn1_ref[...])
    hn = jnp.dot(t, wn2_ref[...],
                 preferred_element_type=jnp.float32) + bn2_ref[...] + h
    h3_ref[...] = hn
    o = _silu(jnp.dot(hn, w1_ref[...],
                      preferred_element_type=jnp.float32) + b1_ref[...])
    o = jnp.dot(o, w2_ref[...], preferred_element_type=jnp.float32) + b2_ref[...]
    gid = lax.broadcasted_iota(jnp.int32, (1, G), 1)
    oh_e = (be_ref[...] == gid).astype(jnp.float32)
    oh_o = (bo_ref[...] == gid).astype(jnp.float32)
    part = (lax.dot_general(oh_e, o[:, :M], (((0,), (0,)), ((), ())),
                            preferred_element_type=jnp.float32)
            + lax.dot_general(oh_o, o[:, M:], (((0,), (0,)), ((), ())),
                              preferred_element_type=jnp.float32))

    @pl.when(i == 0)
    def _():
        acc_ref[...] = jnp.zeros_like(acc_ref)

    acc_ref[...] += part

    @pl.when(i == GRID_N - 1)
    def _():
        og = _silu(jnp.dot(acc_ref[...], w3_ref[...],
                           preferred_element_type=jnp.float32) + b3_ref[...])
        fin_ref[...] = jnp.dot(og, w4_ref[...],
                               preferred_element_type=jnp.float32) + b4_ref[...]


def _row_spec(rows, cols):
    return pl.BlockSpec((rows, cols), lambda i: (i, 0))


def _full_spec(shape):
    nd = len(shape)
    return pl.BlockSpec(shape, lambda i: (0,) * nd)


_N2 = N_PAD // 2
_BN2 = BN // 2
_DP = 2 * M


def _tck_in(x2, wp2, bp2, wa2, ba2, wb2):
    return pl.pallas_call(
        _tck_in_body,
        grid=(GRID_N,),
        in_specs=[_row_spec(_BN2, 2 * F_IN), _full_spec((2 * F_IN, _DP)),
                  _full_spec((1, _DP)), _full_spec((_DP, _DP)),
                  _full_spec((1, _DP)), _full_spec((_DP, _DP))],
        out_specs=[_row_spec(_BN2, _DP), _row_spec(_BN2, _DP),
                   _row_spec(_BN2, _DP)],
        out_shape=[jax.ShapeDtypeStruct((_N2, _DP), jnp.float32),
                   jax.ShapeDtypeStruct((_N2, _DP), jnp.float32),
                   jax.ShapeDtypeStruct((_N2, _DP), jnp.float32)],
    )(x2, wp2, bp2, wa2, ba2, wb2)


def _tck_mid(h2, mi2, wn1a2, wn1b2, bn12, wn22, bn22, wa2, ba2, wb2):
    return pl.pallas_call(
        _tck_mid_body,
        grid=(GRID_N,),
        in_specs=[_row_spec(_BN2, _DP), _row_spec(_BN2, _DP)]
        + [_full_spec((_DP, _DP)), _full_spec((_DP, _DP)),
           _full_spec((1, _DP)), _full_spec((_DP, _DP)),
           _full_spec((1, _DP)), _full_spec((_DP, _DP)),
           _full_spec((1, _DP)), _full_spec((_DP, _DP))],
        out_specs=[_row_spec(_BN2, _DP), _row_spec(_BN2, _DP),
                   _row_spec(_BN2, _DP)],
        out_shape=[jax.ShapeDtypeStruct((_N2, _DP), jnp.float32),
                   jax.ShapeDtypeStruct((_N2, _DP), jnp.float32),
                   jax.ShapeDtypeStruct((_N2, _DP), jnp.float32)],
    )(h2, mi2, wn1a2, wn1b2, bn12, wn22, bn22, wa2, ba2, wb2)


def _tck_edge(p2, w2x2, b2x2):
    return pl.pallas_call(
        _tck_edge_body,
        grid=(GRID_E,),
        in_specs=[_row_spec(BE // 2, 2 * M), _full_spec((2 * M, 2 * M)),
                  _full_spec((1, 2 * M))],
        out_specs=_row_spec(BE // 2, 2 * M),
        out_shape=jax.ShapeDtypeStruct((E_PAD // 2, 2 * M), jnp.float32),
    )(p2, w2x2, b2x2)


def _tck_last(h2, mi2, be, bo, wn1a2, wn1b2, bn12, wn22, bn22,
              w12, b12, w22, b22, w3, b3, w4, b4):
    return pl.pallas_call(
        _tck_last_body,
        grid=(GRID_N,),
        in_specs=[_row_spec(_BN2, _DP), _row_spec(_BN2, _DP),
                  _row_spec(_BN2, 1), _row_spec(_BN2, 1),
                  _full_spec((_DP, _DP)), _full_spec((_DP, _DP)),
                  _full_spec((1, _DP)), _full_spec((_DP, _DP)),
                  _full_spec((1, _DP)), _full_spec((_DP, _DP)),
                  _full_spec((1, _DP)), _full_spec((_DP, _DP)),
                  _full_spec((1, _DP)),
                  _full_spec((D, M)), _full_spec((1, M)),
                  _full_spec((M, 1)), _full_spec((1, 1))],
        out_specs=[_row_spec(_BN2, _DP), _full_spec((G, 1))],
        out_shape=[jax.ShapeDtypeStruct((_N2, _DP), jnp.float32),
                   jax.ShapeDtypeStruct((G, 1), jnp.float32)],
        scratch_shapes=[pltpu.VMEM((G, D), jnp.float32)],
    )(h2, mi2, be, bo, wn1a2, wn1b2, bn12, wn22, bn22,
      w12, b12, w22, b22, w3, b3, w4, b4)


_NGG = 2
_GATHER_ITERS = GPW // (2 * _NGG)
_CG = _NGG * 128

_NGS = 2
_EPS = E_PAD // 16
_GPS = _EPS // 128
_SCATTER_ITERS = _GPS // (2 * _NGS)
_CS = _NGS * 128
_HD = D // 2
_ZR = 416
_NCHUNK = N_PAD // _ZR


@functools.lru_cache(maxsize=None)
def _make_sck_gather():
    mesh = plsc.VectorSubcoreMesh(core_axis_name="c", subcore_axis_name="s")
    return functools.partial(
        pl.kernel,
        out_type=jax.ShapeDtypeStruct((E_PAD, M), jnp.float32),
        mesh=mesh,
        scratch_types=[
            pltpu.VMEM((2 * _NGG, 128), jnp.int32),
            pltpu.VMEM((2 * _NGG, 128), jnp.int32),
            pltpu.VMEM((2 * _CG, M), jnp.float32),
            pltpu.VMEM((2 * _CG, M), jnp.float32),
            pltpu.SemaphoreType.DMA,
            pltpu.SemaphoreType.DMA,
            pltpu.SemaphoreType.DMA,
            pltpu.SemaphoreType.DMA,
            pltpu.SemaphoreType.DMA,
            pltpu.SemaphoreType.DMA,
        ],
        compiler_params=pltpu.CompilerParams(use_tc_tiling_on_sc=False),
    )(_sck_gather_body)


def _sck_gather(A, B, dst_p, src_p):
    return _make_sck_gather()(A, B, dst_p, src_p)


def _sck_gather_body(a_hbm, b_hbm, dst_hbm, src_hbm, p_hbm,
                     idxd_v, idxs_v, bufa_v, bufb_v,
                     is_e, is_o, g_e, g_o, w_e, w_o):
    wid = lax.axis_index("s") * 2 + lax.axis_index("c")
    gbase = wid * GPW

    def fire_idx(pair):
        goff_e = pl.multiple_of(gbase + (2 * pair) * _NGG, _NGG)
        goff_o = pl.multiple_of(gbase + (2 * pair + 1) * _NGG, _NGG)
        pltpu.async_copy(dst_hbm.at[pl.ds(goff_e, _NGG)],
                         idxd_v.at[pl.ds(0, _NGG)], is_e)
        pltpu.async_copy(src_hbm.at[pl.ds(goff_e, _NGG)],
                         idxs_v.at[pl.ds(0, _NGG)], is_e)
        pltpu.async_copy(dst_hbm.at[pl.ds(goff_o, _NGG)],
                         idxd_v.at[pl.ds(_NGG, _NGG)], is_o)
        pltpu.async_copy(src_hbm.at[pl.ds(goff_o, _NGG)],
                         idxs_v.at[pl.ds(_NGG, _NGG)], is_o)

    def wait_idx():
        pltpu.make_async_copy(dst_hbm.at[pl.ds(0, _NGG)],
                              idxd_v.at[pl.ds(0, _NGG)], is_e).wait()
        pltpu.make_async_copy(src_hbm.at[pl.ds(0, _NGG)],
                              idxs_v.at[pl.ds(0, _NGG)], is_e).wait()
        pltpu.make_async_copy(dst_hbm.at[pl.ds(0, _NGG)],
                              idxd_v.at[pl.ds(_NGG, _NGG)], is_o).wait()
        pltpu.make_async_copy(src_hbm.at[pl.ds(0, _NGG)],
                              idxs_v.at[pl.ds(_NGG, _NGG)], is_o).wait()

    def wait_writes():
        pltpu.make_async_copy(bufa_v.at[pl.ds(0, _CG)],
                              p_hbm.at[pl.ds(0, _CG)], w_e).wait()
        pltpu.make_async_copy(bufa_v.at[pl.ds(_CG, _CG)],
                              p_hbm.at[pl.ds(_CG, _CG)], w_o).wait()

    fire_idx(0)

    def iteration(k, carry):
        goff_e = pl.multiple_of(gbase + (2 * k) * _NGG, _NGG)
        goff_o = pl.multiple_of(gbase + (2 * k + 1) * _NGG, _NGG)
        eoff_e = pl.multiple_of(goff_e * 128, _CG)
        eoff_o = pl.multiple_of(goff_o * 128, _CG)

        @pl.when(k > 0)
        def _():
            wait_writes()

        wait_idx()
        dg_e = []
        for j in range(_NGG):
            dg_e.append(pltpu.async_copy(
                a_hbm.at[idxd_v.at[j]],
                bufa_v.at[pl.ds(j * 128, 128)], g_e))
            dg_e.append(pltpu.async_copy(
                b_hbm.at[idxs_v.at[j]],
                bufb_v.at[pl.ds(j * 128, 128)], g_e))
        dg_o = []
        for j in range(_NGG, 2 * _NGG):
            dg_o.append(pltpu.async_copy(
                a_hbm.at[idxd_v.at[j]],
                bufa_v.at[pl.ds(j * 128, 128)], g_o))
            dg_o.append(pltpu.async_copy(
                b_hbm.at[idxs_v.at[j]],
                bufb_v.at[pl.ds(j * 128, 128)], g_o))

        def add_rows(base):
            def add_row(r, c):
                for kk in range(M // 16):
                    sl = pl.ds(kk * 16, 16)
                    bufa_v[base + r, sl] += bufb_v[base + r, sl]
                return c
            lax.fori_loop(0, _CG, add_row, 0, unroll=4)

        for d in dg_e:
            d.wait()
        add_rows(0)
        pltpu.async_copy(bufa_v.at[pl.ds(0, _CG)],
                         p_hbm.at[pl.ds(eoff_e, _CG)], w_e)
        for d in dg_o:
            d.wait()

        @pl.when(k + 1 < _GATHER_ITERS)
        def _():
            fire_idx(k + 1)

        add_rows(_CG)
        pltpu.async_copy(bufa_v.at[pl.ds(_CG, _CG)],
                         p_hbm.at[pl.ds(eoff_o, _CG)], w_o)
        return carry

    lax.fori_loop(0, _GATHER_ITERS, iteration, 0)
    wait_writes()


@functools.lru_cache(maxsize=None)
def _make_sck_scatter():
    mesh = plsc.VectorSubcoreMesh(core_axis_name="c", subcore_axis_name="s")
    return functools.partial(
        pl.kernel,
        out_type=jax.ShapeDtypeStruct((N_PAD, D), jnp.float32),
        mesh=mesh,
        scratch_types=[
            pltpu.VMEM((2 * _NGS, 128), jnp.int32),
            pltpu.VMEM((2 * _CS, _HD), jnp.float32),
            pltpu.VMEM_SHARED((N_PAD, _HD), jnp.float32),
            pltpu.SemaphoreType.DMA,
            pltpu.SemaphoreType.DMA,
            pltpu.SemaphoreType.DMA,
            pltpu.SemaphoreType.DMA,
            pltpu.SemaphoreType.DMA,
            pltpu.SemaphoreType.DMA,
        ],
        compiler_params=pltpu.CompilerParams(use_tc_tiling_on_sc=False),
    )(_sck_scatter_body)


def _sck_scatter(m, dst_p):
    return _make_sck_scatter()(m, dst_p)


def _sck_scatter_body(m_hbm, dst_hbm, mi_hbm, idx_v, rows_v, acc_sh,
                      is_e, is_o, m_e, m_o, ad_e, ad_o):
    c = lax.axis_index("c")
    s = lax.axis_index("s")
    col0 = c * _HD

    def zrow(r, carry):
        for k in range(_HD // 16):
            rows_v[r, pl.ds(k * 16, 16)] = jnp.zeros((16,), jnp.float32)
        return carry

    lax.fori_loop(0, _ZR, zrow, 0)
    for k in range(8):
        cid = s + 16 * k

        @pl.when(cid < _NCHUNK)
        def _():
            pltpu.sync_copy(rows_v.at[pl.ds(0, _ZR)],
                            acc_sh.at[pl.ds(cid * _ZR, _ZR)])

    plsc.subcore_barrier()

    gbase = s * _GPS

    def iteration(k, carry):
        goff_e = pl.multiple_of(gbase + (2 * k) * _NGS, _NGS)
        goff_o = pl.multiple_of(gbase + (2 * k + 1) * _NGS, _NGS)
        eoff_e = pl.multiple_of(goff_e * 128, _CS)
        eoff_o = pl.multiple_of(goff_o * 128, _CS)
        di_e = pltpu.async_copy(dst_hbm.at[pl.ds(goff_e, _NGS)],
                                idx_v.at[pl.ds(0, _NGS)], is_e)
        dm_e = pltpu.async_copy(
            m_hbm.at[pl.ds(eoff_e, _CS), pl.ds(col0, _HD)],
            rows_v.at[pl.ds(0, _CS)], m_e)
        di_o = pltpu.async_copy(dst_hbm.at[pl.ds(goff_o, _NGS)],
                                idx_v.at[pl.ds(_NGS, _NGS)], is_o)
        dm_o = pltpu.async_copy(
            m_hbm.at[pl.ds(eoff_o, _CS), pl.ds(col0, _HD)],
            rows_v.at[pl.ds(_CS, _CS)], m_o)
        di_e.wait()
        dm_e.wait()
        da_e = []
        for j in range(_NGS):
            d = pltpu.make_async_copy(rows_v.at[pl.ds(j * 128, 128)],
                                      acc_sh.at[idx_v.at[j]], ad_e)
            d.start(add=True)
            da_e.append(d)
        di_o.wait()
        dm_o.wait()
        da_o = []
        for j in range(_NGS, 2 * _NGS):
            d = pltpu.make_async_copy(rows_v.at[pl.ds(j * 128, 128)],
                                      acc_sh.at[idx_v.at[j]], ad_o)
            d.start(add=True)
            da_o.append(d)
        for d in da_e + da_o:
            d.wait()
        return carry

    lax.fori_loop(0, _SCATTER_ITERS, iteration, 0)
    plsc.subcore_barrier()

    for k in range(8):
        cid = s + 16 * k

        @pl.when(cid < _NCHUNK)
        def _():
            pltpu.sync_copy(acc_sh.at[pl.ds(cid * _ZR, _ZR)],
                            mi_hbm.at[pl.ds(cid * _ZR, _ZR),
                                      pl.ds(col0, _HD)])


def kernel(x, pos, edge_index, batch, Wp, bp, We1, be1, We2, be2,
           Wn1, bn1, Wn2, bn2, W1, b1, W2, b2, W3, b3, W4, b4):
    f32 = jnp.float32

    def bd(w):
        z = jnp.zeros(w.shape, f32)
        return jnp.concatenate([
            jnp.concatenate([w, z], axis=-1),
            jnp.concatenate([z, w], axis=-1)], axis=-2)

    def dup(b):
        return jnp.concatenate([b, b], axis=-1)[..., None, :]

    x_p = jnp.pad(x, ((0, N_PAD - N), (0, 0)))
    x2 = x_p.reshape(_N2, 2 * F_IN)
    batch_p = jnp.pad(batch, (0, N_PAD - N), constant_values=G)
    be_ = batch_p[0::2].reshape(_N2, 1)
    bo_ = batch_p[1::2].reshape(_N2, 1)
    npad = E_PAD - E
    pad_rows = N + (jnp.arange(npad, dtype=jnp.int32) % 32)
    src_p = jnp.concatenate([edge_index[0], pad_rows]).reshape(N_GROUPS, 128)
    dst_p = jnp.concatenate([edge_index[1], pad_rows]).reshape(N_GROUPS, 128)

    Wa2 = bd(We1[:, :D, :])
    Wb2 = bd(We1[:, D:, :])
    Wn1a2 = bd(Wn1[:, :D, :])
    Wn1b2 = bd(Wn1[:, D:, :])
    Wn22 = bd(Wn2)
    w2x2 = bd(We2)
    Wp2 = bd(Wp)
    W12 = bd(W1)
    W22 = bd(W2)
    ba2 = dup(be1)
    be2_2x = dup(be2)
    bn12 = dup(bn1)
    bn22 = dup(bn2)
    bp2 = dup(bp)
    b12 = dup(b1)
    b22 = dup(b2)
    b3_2 = b3.reshape(1, M)
    b4_2 = b4.reshape(1, 1)

    h2, A2, B2 = _tck_in(x2, Wp2, bp2, Wa2[0], ba2[0], Wb2[0])
    for l in range(L):
        P = _sck_gather(A2.reshape(N_PAD, M), B2.reshape(N_PAD, M),
                        dst_p, src_p)
        m2 = _tck_edge(P.reshape(E_PAD // 2, 2 * M), w2x2[l], be2_2x[l])
        mi = _sck_scatter(m2.reshape(E_PAD, M), dst_p)
        mi2 = mi.reshape(_N2, _DP)
        if l + 1 < L:
            h2, A2, B2 = _tck_mid(h2, mi2, Wn1a2[l], Wn1b2[l], bn12[l],
                                  Wn22[l], bn22[l], Wa2[l + 1], ba2[l + 1],
                                  Wb2[l + 1])
        else:
            h3_2, fin = _tck_last(h2, mi2, be_, bo_, Wn1a2[l], Wn1b2[l],
                                  bn12[l], Wn22[l], bn22[l], W12, b12,
                                  W22, b22, W3, b3_2, W4, b4_2)
    h3 = h3_2.reshape(N_PAD, D)[:N]
    return (fin.reshape(-1), h3.astype(f32))


# speedup vs baseline: 1.2793x; 1.0003x over previous
import functools

import jax
import jax.numpy as jnp
from jax import lax
from jax.experimental import pallas as pl
from jax.experimental.pallas import tpu as pltpu
from jax.experimental.pallas import tpu_sc as plsc

N = 50000
E = 800000
F_IN = 14
D = 64
M = 64
L = 3
G = 64

N_PAD = 52000
E_PAD = 819200
N_GROUPS = E_PAD // 128
NW = 32
GPW = N_GROUPS // NW
BN = 2000
GRID_N = N_PAD // BN
BE = 8192
GRID_E = E_PAD // BE


def _silu(x):
    return x * jax.nn.sigmoid(x)


def _tck_in_body(x_ref, wp_ref, bp_ref, wa_ref, ba_ref, wb_ref,
                 h_ref, a_ref, b_ref):
    h = jnp.dot(x_ref[...], wp_ref[...],
                preferred_element_type=jnp.float32) + bp_ref[...]
    h_ref[...] = h
    a_ref[...] = jnp.dot(h, wa_ref[...],
                         preferred_element_type=jnp.float32) + ba_ref[...]
    b_ref[...] = jnp.dot(h, wb_ref[...], preferred_element_type=jnp.float32)


def _tck_mid_body(h_ref, mi_ref, wn1a_ref, wn1b_ref, bn1_ref, wn2_ref,
                  bn2_ref, wa_ref, ba_ref, wb_ref, hn_ref, a_ref, b_ref):
    h = h_ref[...]
    t = _silu(jnp.dot(h, wn1a_ref[...], preferred_element_type=jnp.float32)
              + jnp.dot(mi_ref[...], wn1b_ref[...],
                        preferred_element_type=jnp.float32)
              + bn1_ref[...])
    hn = jnp.dot(t, wn2_ref[...],
                 preferred_element_type=jnp.float32) + bn2_ref[...] + h
    hn_ref[...] = hn
    a_ref[...] = jnp.dot(hn, wa_ref[...],
                         preferred_element_type=jnp.float32) + ba_ref[...]
    b_ref[...] = jnp.dot(hn, wb_ref[...], preferred_element_type=jnp.float32)


def _tck_edge_body(p_ref, w_ref, b_ref, m_ref):
    t = _silu(p_ref[...])
    m_ref[...] = _silu(jnp.dot(t, w_ref[...],
                               preferred_element_type=jnp.float32) + b_ref[...])


def _tck_last_body(h_ref, mi_ref, be_ref, bo_ref, wn1a_ref, wn1b_ref,
                   bn1_ref, wn2_ref, bn2_ref, w1_ref, b1_ref, w2_ref, b2_ref,
                   w3_ref, b3_ref, w4_ref, b4_ref,
                   h3_ref, fin_ref, acc_ref):
    i = pl.program_id(0)
    h = h_ref[...]
    t = _silu(jnp.dot(h, wn1a_ref[...], preferred_element_type=jnp.float32)
              + jnp.dot(mi_ref[...], wn1b_ref[...],
                        preferred_element_type=jnp.float32)
              + bn1_ref[...])
    hn = jnp.dot(t, wn2_ref[...],
                 preferred_element_type=jnp.float32) + bn2_ref[...] + h
    h3_ref[...] = hn
    o = _silu(jnp.dot(hn, w1_ref[...],
                      preferred_element_type=jnp.float32) + b1_ref[...])
    o = jnp.dot(o, w2_ref[...], preferred_element_type=jnp.float32) + b2_ref[...]
    gid = lax.broadcasted_iota(jnp.int32, (1, G), 1)
    oh_e = (be_ref[...] == gid).astype(jnp.float32)
    oh_o = (bo_ref[...] == gid).astype(jnp.float32)
    part = (lax.dot_general(oh_e, o[:, :M], (((0,), (0,)), ((), ())),
                            preferred_element_type=jnp.float32)
            + lax.dot_general(oh_o, o[:, M:], (((0,), (0,)), ((), ())),
                              preferred_element_type=jnp.float32))

    @pl.when(i == 0)
    def _():
        acc_ref[...] = jnp.zeros_like(acc_ref)

    acc_ref[...] += part

    @pl.when(i == GRID_N - 1)
    def _():
        og = _silu(jnp.dot(acc_ref[...], w3_ref[...],
                           preferred_element_type=jnp.float32) + b3_ref[...])
        fin_ref[...] = jnp.dot(og, w4_ref[...],
                               preferred_element_type=jnp.float32) + b4_ref[...]


def _row_spec(rows, cols):
    return pl.BlockSpec((rows, cols), lambda i: (i, 0))


def _full_spec(shape):
    nd = len(shape)
    return pl.BlockSpec(shape, lambda i: (0,) * nd)


_N2 = N_PAD // 2
_BN2 = BN // 2
_DP = 2 * M


def _tck_in(x2, wp2, bp2, wa2, ba2, wb2):
    return pl.pallas_call(
        _tck_in_body,
        grid=(GRID_N,),
        in_specs=[_row_spec(_BN2, 2 * F_IN), _full_spec((2 * F_IN, _DP)),
                  _full_spec((1, _DP)), _full_spec((_DP, _DP)),
                  _full_spec((1, _DP)), _full_spec((_DP, _DP))],
        out_specs=[_row_spec(_BN2, _DP), _row_spec(_BN2, _DP),
                   _row_spec(_BN2, _DP)],
        out_shape=[jax.ShapeDtypeStruct((_N2, _DP), jnp.float32),
                   jax.ShapeDtypeStruct((_N2, _DP), jnp.float32),
                   jax.ShapeDtypeStruct((_N2, _DP), jnp.float32)],
    )(x2, wp2, bp2, wa2, ba2, wb2)


def _tck_mid(h2, mi2, wn1a2, wn1b2, bn12, wn22, bn22, wa2, ba2, wb2):
    return pl.pallas_call(
        _tck_mid_body,
        grid=(GRID_N,),
        in_specs=[_row_spec(_BN2, _DP), _row_spec(_BN2, _DP)]
        + [_full_spec((_DP, _DP)), _full_spec((_DP, _DP)),
           _full_spec((1, _DP)), _full_spec((_DP, _DP)),
           _full_spec((1, _DP)), _full_spec((_DP, _DP)),
           _full_spec((1, _DP)), _full_spec((_DP, _DP))],
        out_specs=[_row_spec(_BN2, _DP), _row_spec(_BN2, _DP),
                   _row_spec(_BN2, _DP)],
        out_shape=[jax.ShapeDtypeStruct((_N2, _DP), jnp.float32),
                   jax.ShapeDtypeStruct((_N2, _DP), jnp.float32),
                   jax.ShapeDtypeStruct((_N2, _DP), jnp.float32)],
    )(h2, mi2, wn1a2, wn1b2, bn12, wn22, bn22, wa2, ba2, wb2)


def _tck_edge(p2, w2x2, b2x2):
    return pl.pallas_call(
        _tck_edge_body,
        grid=(GRID_E,),
        in_specs=[_row_spec(BE // 2, 2 * M), _full_spec((2 * M, 2 * M)),
                  _full_spec((1, 2 * M))],
        out_specs=_row_spec(BE // 2, 2 * M),
        out_shape=jax.ShapeDtypeStruct((E_PAD // 2, 2 * M), jnp.float32),
    )(p2, w2x2, b2x2)


def _tck_last(h2, mi2, be, bo, wn1a2, wn1b2, bn12, wn22, bn22,
              w12, b12, w22, b22, w3, b3, w4, b4):
    return pl.pallas_call(
        _tck_last_body,
        grid=(GRID_N,),
        in_specs=[_row_spec(_BN2, _DP), _row_spec(_BN2, _DP),
                  _row_spec(_BN2, 1), _row_spec(_BN2, 1),
                  _full_spec((_DP, _DP)), _full_spec((_DP, _DP)),
                  _full_spec((1, _DP)), _full_spec((_DP, _DP)),
                  _full_spec((1, _DP)), _full_spec((_DP, _DP)),
                  _full_spec((1, _DP)), _full_spec((_DP, _DP)),
                  _full_spec((1, _DP)),
                  _full_spec((D, M)), _full_spec((1, M)),
                  _full_spec((M, 1)), _full_spec((1, 1))],
        out_specs=[_row_spec(_BN2, _DP), _full_spec((G, 1))],
        out_shape=[jax.ShapeDtypeStruct((_N2, _DP), jnp.float32),
                   jax.ShapeDtypeStruct((G, 1), jnp.float32)],
        scratch_shapes=[pltpu.VMEM((G, D), jnp.float32)],
    )(h2, mi2, be, bo, wn1a2, wn1b2, bn12, wn22, bn22,
      w12, b12, w22, b22, w3, b3, w4, b4)


_NGG = 2
_GATHER_ITERS = GPW // (2 * _NGG)
_CG = _NGG * 128

_NGS = 2
_EPS = E_PAD // 16
_GPS = _EPS // 128
_SCATTER_ITERS = _GPS // (2 * _NGS)
_CS = _NGS * 128
_HD = D // 2
_ZR = 416
_NCHUNK = N_PAD // _ZR


@functools.lru_cache(maxsize=None)
def _make_sck_gather():
    mesh = plsc.VectorSubcoreMesh(core_axis_name="c", subcore_axis_name="s")
    return functools.partial(
        pl.kernel,
        out_type=jax.ShapeDtypeStruct((E_PAD, M), jnp.float32),
        mesh=mesh,
        scratch_types=[
            pltpu.VMEM((2 * _NGG, 128), jnp.int32),
            pltpu.VMEM((2 * _NGG, 128), jnp.int32),
            pltpu.VMEM((2 * _CG, M), jnp.float32),
            pltpu.VMEM((2 * _CG, M), jnp.float32),
            pltpu.SemaphoreType.DMA,
            pltpu.SemaphoreType.DMA,
            pltpu.SemaphoreType.DMA,
            pltpu.SemaphoreType.DMA,
            pltpu.SemaphoreType.DMA,
            pltpu.SemaphoreType.DMA,
        ],
        compiler_params=pltpu.CompilerParams(use_tc_tiling_on_sc=False),
    )(_sck_gather_body)


def _sck_gather(A, B, dst_p, src_p):
    return _make_sck_gather()(A, B, dst_p, src_p)


def _sck_gather_body(a_hbm, b_hbm, dst_hbm, src_hbm, p_hbm,
                     idxd_v, idxs_v, bufa_v, bufb_v,
                     is_e, is_o, g_e, g_o, w_e, w_o):
    wid = lax.axis_index("s") * 2 + lax.axis_index("c")
    gbase = wid * GPW

    def fire_idx(pair):
        goff_e = pl.multiple_of(gbase + (2 * pair) * _NGG, _NGG)
        goff_o = pl.multiple_of(gbase + (2 * pair + 1) * _NGG, _NGG)
        pltpu.async_copy(dst_hbm.at[pl.ds(goff_e, _NGG)],
                         idxd_v.at[pl.ds(0, _NGG)], is_e)
        pltpu.async_copy(src_hbm.at[pl.ds(goff_e, _NGG)],
                         idxs_v.at[pl.ds(0, _NGG)], is_e)
        pltpu.async_copy(dst_hbm.at[pl.ds(goff_o, _NGG)],
                         idxd_v.at[pl.ds(_NGG, _NGG)], is_o)
        pltpu.async_copy(src_hbm.at[pl.ds(goff_o, _NGG)],
                         idxs_v.at[pl.ds(_NGG, _NGG)], is_o)

    def wait_idx():
        pltpu.make_async_copy(dst_hbm.at[pl.ds(0, _NGG)],
                              idxd_v.at[pl.ds(0, _NGG)], is_e).wait()
        pltpu.make_async_copy(src_hbm.at[pl.ds(0, _NGG)],
                              idxs_v.at[pl.ds(0, _NGG)], is_e).wait()
        pltpu.make_async_copy(dst_hbm.at[pl.ds(0, _NGG)],
                              idxd_v.at[pl.ds(_NGG, _NGG)], is_o).wait()
        pltpu.make_async_copy(src_hbm.at[pl.ds(0, _NGG)],
                              idxs_v.at[pl.ds(_NGG, _NGG)], is_o).wait()

    def wait_writes():
        pltpu.make_async_copy(bufa_v.at[pl.ds(0, _CG)],
                              p_hbm.at[pl.ds(0, _CG)], w_e).wait()
        pltpu.make_async_copy(bufa_v.at[pl.ds(_CG, _CG)],
                              p_hbm.at[pl.ds(_CG, _CG)], w_o).wait()

    fire_idx(0)

    def iteration(k, carry):
        goff_e = pl.multiple_of(gbase + (2 * k) * _NGG, _NGG)
        goff_o = pl.multiple_of(gbase + (2 * k + 1) * _NGG, _NGG)
        eoff_e = pl.multiple_of(goff_e * 128, _CG)
        eoff_o = pl.multiple_of(goff_o * 128, _CG)

        @pl.when(k > 0)
        def _():
            wait_writes()

        wait_idx()
        dg_e = []
        for j in range(_NGG):
            dg_e.append(pltpu.async_copy(
                a_hbm.at[idxd_v.at[j]],
                bufa_v.at[pl.ds(j * 128, 128)], g_e))
            dg_e.append(pltpu.async_copy(
                b_hbm.at[idxs_v.at[j]],
                bufb_v.at[pl.ds(j * 128, 128)], g_e))
        dg_o = []
        for j in range(_NGG, 2 * _NGG):
            dg_o.append(pltpu.async_copy(
                a_hbm.at[idxd_v.at[j]],
                bufa_v.at[pl.ds(j * 128, 128)], g_o))
            dg_o.append(pltpu.async_copy(
                b_hbm.at[idxs_v.at[j]],
                bufb_v.at[pl.ds(j * 128, 128)], g_o))

        def add_rows(base):
            def add_row(r, c):
                for kk in range(M // 16):
                    sl = pl.ds(kk * 16, 16)
                    bufa_v[base + r, sl] += bufb_v[base + r, sl]
                return c
            lax.fori_loop(0, _CG, add_row, 0, unroll=8)

        for d in dg_e:
            d.wait()
        add_rows(0)
        pltpu.async_copy(bufa_v.at[pl.ds(0, _CG)],
                         p_hbm.at[pl.ds(eoff_e, _CG)], w_e)
        for d in dg_o:
            d.wait()

        @pl.when(k + 1 < _GATHER_ITERS)
        def _():
            fire_idx(k + 1)

        add_rows(_CG)
        pltpu.async_copy(bufa_v.at[pl.ds(_CG, _CG)],
                         p_hbm.at[pl.ds(eoff_o, _CG)], w_o)
        return carry

    lax.fori_loop(0, _GATHER_ITERS, iteration, 0)
    wait_writes()


@functools.lru_cache(maxsize=None)
def _make_sck_scatter():
    mesh = plsc.VectorSubcoreMesh(core_axis_name="c", subcore_axis_name="s")
    return functools.partial(
        pl.kernel,
        out_type=jax.ShapeDtypeStruct((N_PAD, D), jnp.float32),
        mesh=mesh,
        scratch_types=[
            pltpu.VMEM((2 * _NGS, 128), jnp.int32),
            pltpu.VMEM((2 * _CS, _HD), jnp.float32),
            pltpu.VMEM_SHARED((N_PAD, _HD), jnp.float32),
            pltpu.SemaphoreType.DMA,
            pltpu.SemaphoreType.DMA,
            pltpu.SemaphoreType.DMA,
            pltpu.SemaphoreType.DMA,
            pltpu.SemaphoreType.DMA,
            pltpu.SemaphoreType.DMA,
        ],
        compiler_params=pltpu.CompilerParams(use_tc_tiling_on_sc=False),
    )(_sck_scatter_body)


def _sck_scatter(m, dst_p):
    return _make_sck_scatter()(m, dst_p)


def _sck_scatter_body(m_hbm, dst_hbm, mi_hbm, idx_v, rows_v, acc_sh,
                      is_e, is_o, m_e, m_o, ad_e, ad_o):
    c = lax.axis_index("c")
    s = lax.axis_index("s")
    col0 = c * _HD

    def zrow(r, carry):
        for k in range(_HD // 16):
            rows_v[r, pl.ds(k * 16, 16)] = jnp.zeros((16,), jnp.float32)
        return carry

    lax.fori_loop(0, _ZR, zrow, 0)
    for k in range(8):
        cid = s + 16 * k

        @pl.when(cid < _NCHUNK)
        def _():
            pltpu.sync_copy(rows_v.at[pl.ds(0, _ZR)],
                            acc_sh.at[pl.ds(cid * _ZR, _ZR)])

    plsc.subcore_barrier()

    gbase = s * _GPS

    def iteration(k, carry):
        goff_e = pl.multiple_of(gbase + (2 * k) * _NGS, _NGS)
        goff_o = pl.multiple_of(gbase + (2 * k + 1) * _NGS, _NGS)
        eoff_e = pl.multiple_of(goff_e * 128, _CS)
        eoff_o = pl.multiple_of(goff_o * 128, _CS)
        di_e = pltpu.async_copy(dst_hbm.at[pl.ds(goff_e, _NGS)],
                                idx_v.at[pl.ds(0, _NGS)], is_e)
        dm_e = pltpu.async_copy(
            m_hbm.at[pl.ds(eoff_e, _CS), pl.ds(col0, _HD)],
            rows_v.at[pl.ds(0, _CS)], m_e)
        di_o = pltpu.async_copy(dst_hbm.at[pl.ds(goff_o, _NGS)],
                                idx_v.at[pl.ds(_NGS, _NGS)], is_o)
        dm_o = pltpu.async_copy(
            m_hbm.at[pl.ds(eoff_o, _CS), pl.ds(col0, _HD)],
            rows_v.at[pl.ds(_CS, _CS)], m_o)
        di_e.wait()
        dm_e.wait()
        da_e = []
        for j in range(_NGS):
            d = pltpu.make_async_copy(rows_v.at[pl.ds(j * 128, 128)],
                                      acc_sh.at[idx_v.at[j]], ad_e)
            d.start(add=True)
            da_e.append(d)
        di_o.wait()
        dm_o.wait()
        da_o = []
        for j in range(_NGS, 2 * _NGS):
            d = pltpu.make_async_copy(rows_v.at[pl.ds(j * 128, 128)],
                                      acc_sh.at[idx_v.at[j]], ad_o)
            d.start(add=True)
            da_o.append(d)
        for d in da_e + da_o:
            d.wait()
        return carry

    lax.fori_loop(0, _SCATTER_ITERS, iteration, 0)
    plsc.subcore_barrier()

    for k in range(8):
        cid = s + 16 * k

        @pl.when(cid < _NCHUNK)
        def _():
            pltpu.sync_copy(acc_sh.at[pl.ds(cid * _ZR, _ZR)],
                            mi_hbm.at[pl.ds(cid * _ZR, _ZR),
                                      pl.ds(col0, _HD)])


def kernel(x, pos, edge_index, batch, Wp, bp, We1, be1, We2, be2,
           Wn1, bn1, Wn2, bn2, W1, b1, W2, b2, W3, b3, W4, b4):
    f32 = jnp.float32

    def bd(w):
        z = jnp.zeros(w.shape, f32)
        return jnp.concatenate([
            jnp.concatenate([w, z], axis=-1),
            jnp.concatenate([z, w], axis=-1)], axis=-2)

    def dup(b):
        return jnp.concatenate([b, b], axis=-1)[..., None, :]

    x_p = jnp.pad(x, ((0, N_PAD - N), (0, 0)))
    x2 = x_p.reshape(_N2, 2 * F_IN)
    batch_p = jnp.pad(batch, (0, N_PAD - N), constant_values=G)
    be_ = batch_p[0::2].reshape(_N2, 1)
    bo_ = batch_p[1::2].reshape(_N2, 1)
    npad = E_PAD - E
    pad_rows = N + (jnp.arange(npad, dtype=jnp.int32) % 32)
    src_p = jnp.concatenate([edge_index[0], pad_rows]).reshape(N_GROUPS, 128)
    dst_p = jnp.concatenate([edge_index[1], pad_rows]).reshape(N_GROUPS, 128)

    Wa2 = bd(We1[:, :D, :])
    Wb2 = bd(We1[:, D:, :])
    Wn1a2 = bd(Wn1[:, :D, :])
    Wn1b2 = bd(Wn1[:, D:, :])
    Wn22 = bd(Wn2)
    w2x2 = bd(We2)
    Wp2 = bd(Wp)
    W12 = bd(W1)
    W22 = bd(W2)
    ba2 = dup(be1)
    be2_2x = dup(be2)
    bn12 = dup(bn1)
    bn22 = dup(bn2)
    bp2 = dup(bp)
    b12 = dup(b1)
    b22 = dup(b2)
    b3_2 = b3.reshape(1, M)
    b4_2 = b4.reshape(1, 1)

    h2, A2, B2 = _tck_in(x2, Wp2, bp2, Wa2[0], ba2[0], Wb2[0])
    for l in range(L):
        P = _sck_gather(A2.reshape(N_PAD, M), B2.reshape(N_PAD, M),
                        dst_p, src_p)
        m2 = _tck_edge(P.reshape(E_PAD // 2, 2 * M), w2x2[l], be2_2x[l])
        mi = _sck_scatter(m2.reshape(E_PAD, M), dst_p)
        mi2 = mi.reshape(_N2, _DP)
        if l + 1 < L:
            h2, A2, B2 = _tck_mid(h2, mi2, Wn1a2[l], Wn1b2[l], bn12[l],
                                  Wn22[l], bn22[l], Wa2[l + 1], ba2[l + 1],
                                  Wb2[l + 1])
        else:
            h3_2, fin = _tck_last(h2, mi2, be_, bo_, Wn1a2[l], Wn1b2[l],
                                  bn12[l], Wn22[l], bn22[l], W12, b12,
                                  W22, b22, W3, b3_2, W4, b4_2)
    h3 = h3_2.reshape(N_PAD, D)[:N]
    return (fin.reshape(-1), h3.astype(f32))
